# baseline (speedup 1.0000x reference)
import functools

import jax
import jax.numpy as jnp
import numpy as np
from jax import lax
from jax.experimental import pallas as pl
from jax.experimental.pallas import tpu as pltpu
from jax.experimental.pallas import tpu_sc as plsc

N_POINTS = 524288
NUM_LEVELS = 16
TBL = 4096
LANES = 16
NW = 32
LV_GROUPS = 2
LV_PER_G = NUM_LEVELS // LV_GROUPS
HALF = LV_PER_G * TBL
CHUNK = N_POINTS // (NW // LV_GROUPS)
BLK = 2048
GW = LV_PER_G * 2
P1 = np.int32(np.uint32(2654435761))
P2 = np.int32(np.uint32(805459861))

_mesh = plsc.VectorSubcoreMesh(core_axis_name="c", subcore_axis_name="s")


@functools.partial(
    pl.kernel,
    mesh=_mesh,
    compiler_params=pltpu.CompilerParams(needs_layout_passes=False),
    out_type=jax.ShapeDtypeStruct((LV_GROUPS * N_POINTS * GW,), jnp.float32),
    scratch_types=[
        pltpu.VMEM((LV_GROUPS * HALF,), jnp.float32),
        pltpu.VMEM((BLK * 3,), jnp.float32),
        pltpu.VMEM((BLK * GW,), jnp.float32),
    ],
)
def _encode_sc(tab_hbm, pos_hbm, out_hbm, tab_v, pos_v, out_v):
    i32 = jnp.int32
    wid = lax.axis_index("s") * 2 + lax.axis_index("c")
    grp = wid & 1
    chunk = wid >> 1

    pltpu.sync_copy(tab_hbm.at[pl.ds(grp * i32(HALF), HALF)],
                    tab_v.at[pl.ds(i32(0), HALF)])
    pltpu.sync_copy(tab_hbm.at[pl.ds(grp * i32(HALF) + i32(2 * HALF), HALF)],
                    tab_v.at[pl.ds(i32(HALF), HALF)])

    iota = lax.iota(jnp.int32, LANES)
    one = jnp.float32(1.0)

    grp0 = grp == 0
    a2s, rm1s = [], []
    for l in range(LV_PER_G):
        a2s.append(jnp.where(grp0, jnp.float32(2.0 ** (l + 3)),
                             jnp.float32(2.0 ** (l + LV_PER_G + 3))))
        rm1s.append(jnp.where(grp0, jnp.int32(16 * 2 ** l - 1),
                              jnp.int32(16 * 2 ** (l + LV_PER_G) - 1)))

    def block_body(t, _):
        base = chunk * i32(CHUNK) + t * i32(BLK)
        pltpu.sync_copy(pos_hbm.at[pl.ds(base * 3, BLK * 3)], pos_v)

        def group_body(j, _):
            rows = j * i32(LANES) + iota
            rows3 = rows * 3
            x = plsc.load_gather(pos_v, [rows3])
            y = plsc.load_gather(pos_v, [rows3 + 1])
            z = plsc.load_gather(pos_v, [rows3 + 2])
            ppx, ppy, ppz = x + one, y + one, z + one
            phx, phy, phz = ppx * 0.5, ppy * 0.5, ppz * 0.5
            rowcol = rows * i32(GW)

            for l in range(LV_PER_G):
                a2 = a2s[l]
                rm1 = rm1s[l]
                lb = i32(l * TBL)

                def axis(pp, ph):
                    A = pp * a2
                    scaled = A - ph
                    ti = scaled.astype(jnp.int32)
                    tf = ti.astype(jnp.float32)
                    w = scaled - tf
                    c1 = jnp.minimum(ti + 1, rm1)
                    return ti, c1, w

                cx0, cx1, wx = axis(ppx, phx)
                cy0, cy1, wy = axis(ppy, phy)
                cz0, cz1, wz = axis(ppz, phz)

                mx0 = (cx0 & 4095) | lb
                mx1 = (cx1 & 4095) | lb
                my0 = (cy0 * P1) & 4095
                my1 = (cy1 * P1) & 4095
                mz0 = (cz0 * P2) & 4095
                mz1 = (cz1 * P2) & 4095

                f = []
                for mx in (mx0, mx1):
                    for my in (my0, my1):
                        for mz in (mz0, mz1):
                            i0 = mx ^ my ^ mz
                            f.append((plsc.load_gather(tab_v, [i0]),
                                      plsc.load_gather(tab_v, [i0 + i32(HALF)])))

                omx, omy, omz = one - wx, one - wy, one - wz
                res = []
                for k in range(2):
                    c00 = f[0][k] * omx + f[1][k] * wx
                    c01 = f[2][k] * omx + f[3][k] * wx
                    c10 = f[4][k] * omx + f[5][k] * wx
                    c11 = f[6][k] * omx + f[7][k] * wx
                    d0 = c00 * omy + c01 * wy
                    d1 = c10 * omy + c11 * wy
                    res.append(d0 * omz + d1 * wz)

                plsc.store_scatter(out_v, [rowcol + i32(2 * l)], res[0])
                plsc.store_scatter(out_v, [rowcol + i32(2 * l + 1)], res[1])
            return i32(0)

        lax.fori_loop(i32(0), i32(BLK // LANES), group_body, i32(0))
        pltpu.sync_copy(out_v,
                        out_hbm.at[pl.ds(grp * i32(N_POINTS * GW) + base * i32(GW),
                                         BLK * GW)])
        return i32(0)

    lax.fori_loop(i32(0), i32(CHUNK // BLK), block_body, i32(0))


_IL_ROWS = 4096


def _interleave_tc(h0_ref, h1_ref, out_ref):
    out_ref[:, 0:GW] = h0_ref[...]
    out_ref[:, GW:2 * GW] = h1_ref[...]


_interleave = pl.pallas_call(
    _interleave_tc,
    grid=(N_POINTS // _IL_ROWS,),
    in_specs=[
        pl.BlockSpec((_IL_ROWS, GW), lambda i: (i, np.int32(0))),
        pl.BlockSpec((_IL_ROWS, GW), lambda i: (i, np.int32(0))),
    ],
    out_specs=pl.BlockSpec((_IL_ROWS, 2 * GW), lambda i: (i, np.int32(0))),
    out_shape=jax.ShapeDtypeStruct((N_POINTS, 2 * GW), jnp.float32),
)


def kernel(positions, tables):
    tabp = jnp.transpose(tables[:, :TBL, :], (2, 0, 1)).reshape(-1)
    tabp = tabp.astype(jnp.float32)
    flat = _encode_sc(tabp, positions.reshape(-1))
    halves = flat.reshape(LV_GROUPS, N_POINTS, GW)
    return _interleave(halves[0], halves[1])


# speedup vs baseline: 1.9169x; 1.9169x over previous
import functools

import jax
import jax.numpy as jnp
import numpy as np
from jax import lax
from jax.experimental import pallas as pl
from jax.experimental.pallas import tpu as pltpu
from jax.experimental.pallas import tpu_sc as plsc

N_POINTS = 524288
NUM_LEVELS = 16
TBL = 4096
LANES = 16
NW = 32
CHUNK = N_POINTS // NW
BLK = 1024
OUTW = NUM_LEVELS * 2
P1 = np.int32(np.uint32(2654435761))
P2 = np.int32(np.uint32(805459861))

_mesh = plsc.VectorSubcoreMesh(core_axis_name="c", subcore_axis_name="s")


@functools.partial(
    pl.kernel,
    mesh=_mesh,
    compiler_params=pltpu.CompilerParams(needs_layout_passes=False),
    out_type=jax.ShapeDtypeStruct((OUTW, N_POINTS), jnp.float32),
    scratch_types=[
        pltpu.VMEM((NUM_LEVELS * TBL,), jnp.int32),
        pltpu.VMEM((BLK * 3,), jnp.float32),
        pltpu.VMEM((OUTW, BLK), jnp.float32),
    ],
)
def _encode_sc(tab_hbm, pos_hbm, out_hbm, tab_v, pos_v, out_v):
    i32 = jnp.int32
    wid = lax.axis_index("s") * 2 + lax.axis_index("c")

    pltpu.sync_copy(tab_hbm, tab_v)

    iota = lax.iota(jnp.int32, LANES)
    one = jnp.float32(1.0)
    himask = i32(np.int32(np.uint32(0xFFFF0000)))

    def block_body(t, _):
        base = wid * i32(CHUNK) + t * i32(BLK)
        pltpu.sync_copy(pos_hbm.at[pl.ds(base * 3, BLK * 3)], pos_v)

        def group_body(j, _):
            j16 = j * i32(LANES)
            rows3 = j16 * 3 + iota * 3
            x = plsc.load_gather(pos_v, [rows3])
            y = plsc.load_gather(pos_v, [rows3 + 1])
            z = plsc.load_gather(pos_v, [rows3 + 2])
            ppx, ppy, ppz = x + one, y + one, z + one
            phx, phy, phz = ppx * 0.5, ppy * 0.5, ppz * 0.5

            for l in range(NUM_LEVELS):
                a2 = jnp.float32(2.0 ** (l + 3))
                rm1 = i32(16 * 2 ** l - 1)
                lb = i32(l * TBL)

                def axis(pp, ph):
                    A = pp * a2
                    scaled = A - ph
                    ti = scaled.astype(jnp.int32)
                    tf = ti.astype(jnp.float32)
                    w = scaled - tf
                    c1 = jnp.minimum(ti + 1, rm1)
                    return ti, c1, w

                cx0, cx1, wx = axis(ppx, phx)
                cy0, cy1, wy = axis(ppy, phy)
                cz0, cz1, wz = axis(ppz, phz)

                mx0 = (cx0 & 4095) | lb
                mx1 = (cx1 & 4095) | lb
                my0 = (cy0 * P1) & 4095
                my1 = (cy1 * P1) & 4095
                mz0 = (cz0 * P2) & 4095
                mz1 = (cz1 * P2) & 4095

                f = []
                for mx in (mx0, mx1):
                    for my in (my0, my1):
                        for mz in (mz0, mz1):
                            wd = plsc.load_gather(tab_v, [mx ^ my ^ mz])
                            f.append((plsc.bitcast(wd & himask, jnp.float32),
                                      plsc.bitcast(wd << 16, jnp.float32)))

                omx, omy, omz = one - wx, one - wy, one - wz
                for k in range(2):
                    c00 = f[0][k] * omx + f[1][k] * wx
                    c01 = f[2][k] * omx + f[3][k] * wx
                    c10 = f[4][k] * omx + f[5][k] * wx
                    c11 = f[6][k] * omx + f[7][k] * wx
                    d0 = c00 * omy + c01 * wy
                    d1 = c10 * omy + c11 * wy
                    out_v[2 * l + k, pl.ds(j16, LANES)] = d0 * omz + d1 * wz
            return i32(0)

        lax.fori_loop(i32(0), i32(BLK // LANES), group_body, i32(0))
        pltpu.sync_copy(out_v, out_hbm.at[:, pl.ds(base, BLK)])
        return i32(0)

    lax.fori_loop(i32(0), i32(CHUNK // BLK), block_body, i32(0))


def kernel(positions, tables):
    t16 = tables[:, :TBL, :].astype(jnp.bfloat16)
    bits = lax.bitcast_convert_type(t16, jnp.uint16).astype(jnp.uint32)
    words = (bits[..., 0] << 16) | bits[..., 1]
    tabw = lax.bitcast_convert_type(words, jnp.int32).reshape(NUM_LEVELS * TBL)
    planes = _encode_sc(tabw, positions.reshape(-1))
    return planes.T


# speedup vs baseline: 4.3694x; 2.2795x over previous
import functools

import jax
import jax.numpy as jnp
import numpy as np
from jax import lax
from jax.experimental import pallas as pl
from jax.experimental.pallas import tpu as pltpu
from jax.experimental.pallas import tpu_sc as plsc

N_POINTS = 524288
NUM_LEVELS = 16
TBL = 4096
LANES = 16
NW = 32
CHUNK = N_POINTS // NW
BLK = 1024
OUTW = NUM_LEVELS * 2
P1 = np.int32(np.uint32(2654435761))
P2 = np.int32(np.uint32(805459861))

_mesh = plsc.VectorSubcoreMesh(core_axis_name="c", subcore_axis_name="s")


@functools.partial(
    pl.kernel,
    mesh=_mesh,
    compiler_params=pltpu.CompilerParams(needs_layout_passes=False),
    out_type=jax.ShapeDtypeStruct((OUTW, N_POINTS), jnp.float32),
    scratch_types=[
        pltpu.VMEM((NUM_LEVELS * TBL,), jnp.int32),
        pltpu.VMEM((BLK,), jnp.float32),
        pltpu.VMEM((BLK,), jnp.float32),
        pltpu.VMEM((BLK,), jnp.float32),
        pltpu.VMEM((OUTW, BLK), jnp.float32),
    ],
)
def _encode_sc(tab_hbm, xs_hbm, ys_hbm, zs_hbm, out_hbm,
               tab_v, x_v, y_v, z_v, out_v):
    i32 = jnp.int32
    wid = lax.axis_index("s") * 2 + lax.axis_index("c")

    pltpu.sync_copy(tab_hbm, tab_v)

    iota = lax.iota(jnp.int32, LANES)
    one = jnp.float32(1.0)
    himask = i32(np.int32(np.uint32(0xFFFF0000)))

    def block_body(t, _):
        base = wid * i32(CHUNK) + t * i32(BLK)
        pltpu.sync_copy(xs_hbm.at[pl.ds(base, BLK)], x_v)
        pltpu.sync_copy(ys_hbm.at[pl.ds(base, BLK)], y_v)
        pltpu.sync_copy(zs_hbm.at[pl.ds(base, BLK)], z_v)

        def group_body(j, _):
            j16 = j * i32(LANES)
            x = x_v[pl.ds(j16, LANES)]
            y = y_v[pl.ds(j16, LANES)]
            z = z_v[pl.ds(j16, LANES)]
            ppx, ppy, ppz = x + one, y + one, z + one
            phx, phy, phz = ppx * 0.5, ppy * 0.5, ppz * 0.5

            for l in range(NUM_LEVELS):
                a2 = jnp.float32(2.0 ** (l + 3))
                rm1 = i32(16 * 2 ** l - 1)
                lb = i32(l * TBL)

                def axis(pp, ph):
                    A = pp * a2
                    scaled = A - ph
                    ti = scaled.astype(jnp.int32)
                    tf = ti.astype(jnp.float32)
                    w = scaled - tf
                    c1 = jnp.minimum(ti + 1, rm1)
                    return ti, c1, w

                cx0, cx1, wx = axis(ppx, phx)
                cy0, cy1, wy = axis(ppy, phy)
                cz0, cz1, wz = axis(ppz, phz)

                mx0 = (cx0 & 4095) | lb
                mx1 = (cx1 & 4095) | lb
                my0 = (cy0 * P1) & 4095
                my1 = (cy1 * P1) & 4095
                mz0 = (cz0 * P2) & 4095
                mz1 = (cz1 * P2) & 4095

                f = []
                for mx in (mx0, mx1):
                    for my in (my0, my1):
                        for mz in (mz0, mz1):
                            wd = plsc.load_gather(tab_v, [mx ^ my ^ mz])
                            f.append((plsc.bitcast(wd & himask, jnp.float32),
                                      plsc.bitcast(wd << 16, jnp.float32)))

                omx, omy, omz = one - wx, one - wy, one - wz
                for k in range(2):
                    c00 = f[0][k] * omx + f[1][k] * wx
                    c01 = f[2][k] * omx + f[3][k] * wx
                    c10 = f[4][k] * omx + f[5][k] * wx
                    c11 = f[6][k] * omx + f[7][k] * wx
                    d0 = c00 * omy + c01 * wy
                    d1 = c10 * omy + c11 * wy
                    out_v[2 * l + k, pl.ds(j16, LANES)] = d0 * omz + d1 * wz
            return i32(0)

        lax.fori_loop(i32(0), i32(BLK // LANES), group_body, i32(0))
        pltpu.sync_copy(out_v, out_hbm.at[:, pl.ds(base, BLK)])
        return i32(0)

    lax.fori_loop(i32(0), i32(CHUNK // BLK), block_body, i32(0))


def kernel(positions, tables):
    t16 = tables[:, :TBL, :].astype(jnp.bfloat16)
    bits = lax.bitcast_convert_type(t16, jnp.uint16).astype(jnp.uint32)
    words = (bits[..., 0] << 16) | bits[..., 1]
    tabw = lax.bitcast_convert_type(words, jnp.int32).reshape(NUM_LEVELS * TBL)
    pt = positions.T
    planes = _encode_sc(tabw, pt[0], pt[1], pt[2])
    return planes.T


# speedup vs baseline: 4.7979x; 1.0981x over previous
import functools

import jax
import jax.numpy as jnp
import numpy as np
from jax import lax
from jax.experimental import pallas as pl
from jax.experimental.pallas import tpu as pltpu
from jax.experimental.pallas import tpu_sc as plsc

N_POINTS = 524288
NUM_LEVELS = 16
TBL = 4096
LANES = 16
NW = 32
CHUNK = N_POINTS // NW
BLK = 512
NBLK = CHUNK // BLK
OUTW = NUM_LEVELS * 2
P1 = np.int32(np.uint32(2654435761))
P2 = np.int32(np.uint32(805459861))

_mesh = plsc.VectorSubcoreMesh(core_axis_name="c", subcore_axis_name="s")


@functools.partial(
    pl.kernel,
    mesh=_mesh,
    compiler_params=pltpu.CompilerParams(needs_layout_passes=False),
    out_type=jax.ShapeDtypeStruct((OUTW, N_POINTS), jnp.float32),
    scratch_types=[
        pltpu.VMEM((NUM_LEVELS * TBL,), jnp.int32),
        pltpu.VMEM((2 * BLK,), jnp.float32),
        pltpu.VMEM((2 * BLK,), jnp.float32),
        pltpu.VMEM((2 * BLK,), jnp.float32),
        pltpu.VMEM((OUTW, 2 * BLK), jnp.float32),
        pltpu.SemaphoreType.DMA,
        pltpu.SemaphoreType.DMA,
        pltpu.SemaphoreType.DMA,
        pltpu.SemaphoreType.DMA,
    ],
)
def _encode_sc(tab_hbm, xs_hbm, ys_hbm, zs_hbm, out_hbm,
               tab_v, x_v, y_v, z_v, out_v,
               isem0, isem1, osem0, osem1):
    i32 = jnp.int32
    wid = lax.axis_index("s") * 2 + lax.axis_index("c")
    row0 = wid * i32(CHUNK)

    pltpu.sync_copy(tab_hbm, tab_v)

    iota = lax.iota(jnp.int32, LANES)
    one = jnp.float32(1.0)
    himask = i32(np.int32(np.uint32(0xFFFF0000)))
    isems = (isem0, isem1)
    osems = (osem0, osem1)

    def start_in(b, base):
        off = i32(b * BLK)
        pltpu.async_copy(xs_hbm.at[pl.ds(base, BLK)], x_v.at[pl.ds(off, BLK)], isems[b])
        pltpu.async_copy(ys_hbm.at[pl.ds(base, BLK)], y_v.at[pl.ds(off, BLK)], isems[b])
        pltpu.async_copy(zs_hbm.at[pl.ds(base, BLK)], z_v.at[pl.ds(off, BLK)], isems[b])

    def wait_in(b, base):
        off = i32(b * BLK)
        pltpu.make_async_copy(xs_hbm.at[pl.ds(base, BLK)], x_v.at[pl.ds(off, BLK)], isems[b]).wait()
        pltpu.make_async_copy(ys_hbm.at[pl.ds(base, BLK)], y_v.at[pl.ds(off, BLK)], isems[b]).wait()
        pltpu.make_async_copy(zs_hbm.at[pl.ds(base, BLK)], z_v.at[pl.ds(off, BLK)], isems[b]).wait()

    def out_buf(b):
        return out_v.at[:, pl.ds(i32(b * BLK), BLK)]

    def out_slice(base):
        return out_hbm.at[:, pl.ds(base, BLK)]

    start_in(0, row0)

    def block_pair(g, _):
        for b in range(2):
            t = g * 2 + i32(b)
            base = row0 + t * i32(BLK)
            wait_in(b, base)
            if b == 0:
                start_in(1, base + i32(BLK))
            else:
                @pl.when(g < i32(NBLK // 2 - 1))
                def _():
                    start_in(0, base + i32(BLK))

            @pl.when(g > 0)
            def _():
                pltpu.make_async_copy(out_buf(b), out_slice(base), osems[b]).wait()

            compute_block(b, base)
            pltpu.async_copy(out_buf(b), out_slice(base), osems[b])
        return i32(0)

    def compute_block(b, base):
        boff = i32(b * BLK)

        def group_body(j, _):
            j16 = boff + j * i32(LANES)
            x = x_v[pl.ds(j16, LANES)]
            y = y_v[pl.ds(j16, LANES)]
            z = z_v[pl.ds(j16, LANES)]
            ppx, ppy, ppz = x + one, y + one, z + one
            phx, phy, phz = ppx * 0.5, ppy * 0.5, ppz * 0.5

            for l in range(NUM_LEVELS):
                a2 = jnp.float32(2.0 ** (l + 3))
                rm1 = i32(16 * 2 ** l - 1)
                lb = i32(l * TBL)

                def axis(pp, ph):
                    A = pp * a2
                    scaled = A - ph
                    ti = scaled.astype(jnp.int32)
                    tf = ti.astype(jnp.float32)
                    w = scaled - tf
                    c1 = jnp.minimum(ti + 1, rm1)
                    return ti, c1, w

                cx0, cx1, wx = axis(ppx, phx)
                cy0, cy1, wy = axis(ppy, phy)
                cz0, cz1, wz = axis(ppz, phz)

                mx0 = (cx0 & 4095) | lb
                mx1 = (cx1 & 4095) | lb
                my0 = (cy0 * P1) & 4095
                my1 = (cy1 * P1) & 4095
                mz0 = (cz0 * P2) & 4095
                mz1 = (cz1 * P2) & 4095

                f = []
                for mx in (mx0, mx1):
                    for my in (my0, my1):
                        for mz in (mz0, mz1):
                            wd = plsc.load_gather(tab_v, [mx ^ my ^ mz])
                            f.append((plsc.bitcast(wd & himask, jnp.float32),
                                      plsc.bitcast(wd << 16, jnp.float32)))

                omx, omy, omz = one - wx, one - wy, one - wz
                for k in range(2):
                    c00 = f[0][k] * omx + f[1][k] * wx
                    c01 = f[2][k] * omx + f[3][k] * wx
                    c10 = f[4][k] * omx + f[5][k] * wx
                    c11 = f[6][k] * omx + f[7][k] * wx
                    d0 = c00 * omy + c01 * wy
                    d1 = c10 * omy + c11 * wy
                    out_v[2 * l + k, pl.ds(j16, LANES)] = d0 * omz + d1 * wz
            return i32(0)

        lax.fori_loop(i32(0), i32(BLK // LANES), group_body, i32(0))

    lax.fori_loop(i32(0), i32(NBLK // 2), block_pair, i32(0))
    last = row0 + i32(CHUNK - 2 * BLK)
    pltpu.make_async_copy(out_buf(0), out_slice(last), osems[0]).wait()
    pltpu.make_async_copy(out_buf(1), out_slice(last + i32(BLK)), osems[1]).wait()


def kernel(positions, tables):
    t16 = tables[:, :TBL, :].astype(jnp.bfloat16)
    bits = lax.bitcast_convert_type(t16, jnp.uint16).astype(jnp.uint32)
    words = (bits[..., 0] << 16) | bits[..., 1]
    tabw = lax.bitcast_convert_type(words, jnp.int32).reshape(NUM_LEVELS * TBL)
    pt = positions.T
    planes = _encode_sc(tabw, pt[0], pt[1], pt[2])
    return planes.T


# speedup vs baseline: 4.9150x; 1.0244x over previous
import functools

import jax
import jax.numpy as jnp
import numpy as np
from jax import lax
from jax.experimental import pallas as pl
from jax.experimental.pallas import tpu as pltpu
from jax.experimental.pallas import tpu_sc as plsc

N_POINTS = 524288
NUM_LEVELS = 16
TBL = 4096
LANES = 16
NW = 32
CHUNK = N_POINTS // NW
BLK = 512
NBLK = CHUNK // BLK
OUTW = NUM_LEVELS * 2
P1 = np.int32(np.uint32(2654435761))
P2 = np.int32(np.uint32(805459861))

_mesh = plsc.VectorSubcoreMesh(core_axis_name="c", subcore_axis_name="s")


@functools.partial(
    pl.kernel,
    mesh=_mesh,
    compiler_params=pltpu.CompilerParams(needs_layout_passes=False),
    out_type=jax.ShapeDtypeStruct((OUTW, N_POINTS), jnp.float32),
    scratch_types=[
        pltpu.VMEM((NUM_LEVELS * TBL,), jnp.int32),
        pltpu.VMEM((2 * BLK,), jnp.float32),
        pltpu.VMEM((2 * BLK,), jnp.float32),
        pltpu.VMEM((2 * BLK,), jnp.float32),
        pltpu.VMEM((OUTW, 2 * BLK), jnp.float32),
        pltpu.SemaphoreType.DMA,
        pltpu.SemaphoreType.DMA,
        pltpu.SemaphoreType.DMA,
        pltpu.SemaphoreType.DMA,
    ],
)
def _encode_sc(tab_hbm, xs_hbm, ys_hbm, zs_hbm, out_hbm,
               tab_v, x_v, y_v, z_v, out_v,
               isem0, isem1, osem0, osem1):
    i32 = jnp.int32
    wid = lax.axis_index("s") * 2 + lax.axis_index("c")
    row0 = wid * i32(CHUNK)

    pltpu.sync_copy(tab_hbm, tab_v)

    iota = lax.iota(jnp.int32, LANES)
    one = jnp.float32(1.0)
    himask = i32(np.int32(np.uint32(0xFFFF0000)))
    isems = (isem0, isem1)
    osems = (osem0, osem1)

    def start_in(b, base):
        off = i32(b * BLK)
        pltpu.async_copy(xs_hbm.at[pl.ds(base, BLK)], x_v.at[pl.ds(off, BLK)], isems[b])
        pltpu.async_copy(ys_hbm.at[pl.ds(base, BLK)], y_v.at[pl.ds(off, BLK)], isems[b])
        pltpu.async_copy(zs_hbm.at[pl.ds(base, BLK)], z_v.at[pl.ds(off, BLK)], isems[b])

    def wait_in(b, base):
        off = i32(b * BLK)
        pltpu.make_async_copy(xs_hbm.at[pl.ds(base, BLK)], x_v.at[pl.ds(off, BLK)], isems[b]).wait()
        pltpu.make_async_copy(ys_hbm.at[pl.ds(base, BLK)], y_v.at[pl.ds(off, BLK)], isems[b]).wait()
        pltpu.make_async_copy(zs_hbm.at[pl.ds(base, BLK)], z_v.at[pl.ds(off, BLK)], isems[b]).wait()

    def out_buf(b):
        return out_v.at[:, pl.ds(i32(b * BLK), BLK)]

    def out_slice(base):
        return out_hbm.at[:, pl.ds(base, BLK)]

    start_in(0, row0)

    def block_pair(g, _):
        for b in range(2):
            t = g * 2 + i32(b)
            base = row0 + t * i32(BLK)
            wait_in(b, base)
            if b == 0:
                start_in(1, base + i32(BLK))
            else:
                @pl.when(g < i32(NBLK // 2 - 1))
                def _():
                    start_in(0, base + i32(BLK))

            @pl.when(g > 0)
            def _():
                pltpu.make_async_copy(out_buf(b), out_slice(base), osems[b]).wait()

            compute_block(b, base)
            pltpu.async_copy(out_buf(b), out_slice(base), osems[b])
        return i32(0)

    def compute_block(b, base):
        boff = i32(b * BLK)

        @plsc.parallel_loop(i32(0), i32(BLK // LANES), i32(1), unroll=2)
        def group_body(j):
            j16 = boff + j * i32(LANES)
            x = x_v[pl.ds(j16, LANES)]
            y = y_v[pl.ds(j16, LANES)]
            z = z_v[pl.ds(j16, LANES)]
            ppx, ppy, ppz = x + one, y + one, z + one
            phx, phy, phz = ppx * 0.5, ppy * 0.5, ppz * 0.5

            for l in range(NUM_LEVELS):
                a2 = jnp.float32(2.0 ** (l + 3))
                rm1 = i32(16 * 2 ** l - 1)
                lb = i32(l * TBL)

                def axis(pp, ph):
                    A = pp * a2
                    scaled = A - ph
                    ti = scaled.astype(jnp.int32)
                    tf = ti.astype(jnp.float32)
                    w = scaled - tf
                    c1 = jnp.minimum(ti + 1, rm1)
                    return ti, c1, w

                cx0, cx1, wx = axis(ppx, phx)
                cy0, cy1, wy = axis(ppy, phy)
                cz0, cz1, wz = axis(ppz, phz)

                mx0 = (cx0 & 4095) | lb
                mx1 = (cx1 & 4095) | lb
                my0 = (cy0 * P1) & 4095
                my1 = (cy1 * P1) & 4095
                mz0 = (cz0 * P2) & 4095
                mz1 = (cz1 * P2) & 4095

                f = []
                for mx in (mx0, mx1):
                    for my in (my0, my1):
                        for mz in (mz0, mz1):
                            wd = plsc.load_gather(tab_v, [mx ^ my ^ mz])
                            f.append((plsc.bitcast(wd & himask, jnp.float32),
                                      plsc.bitcast(wd << 16, jnp.float32)))

                omx, omy, omz = one - wx, one - wy, one - wz
                for k in range(2):
                    c00 = f[0][k] * omx + f[1][k] * wx
                    c01 = f[2][k] * omx + f[3][k] * wx
                    c10 = f[4][k] * omx + f[5][k] * wx
                    c11 = f[6][k] * omx + f[7][k] * wx
                    d0 = c00 * omy + c01 * wy
                    d1 = c10 * omy + c11 * wy
                    out_v[2 * l + k, pl.ds(j16, LANES)] = d0 * omz + d1 * wz

    lax.fori_loop(i32(0), i32(NBLK // 2), block_pair, i32(0))
    last = row0 + i32(CHUNK - 2 * BLK)
    pltpu.make_async_copy(out_buf(0), out_slice(last), osems[0]).wait()
    pltpu.make_async_copy(out_buf(1), out_slice(last + i32(BLK)), osems[1]).wait()


def kernel(positions, tables):
    t16 = tables[:, :TBL, :].astype(jnp.bfloat16)
    bits = lax.bitcast_convert_type(t16, jnp.uint16).astype(jnp.uint32)
    words = (bits[..., 0] << 16) | bits[..., 1]
    tabw = lax.bitcast_convert_type(words, jnp.int32).reshape(NUM_LEVELS * TBL)
    pt = positions.T
    planes = _encode_sc(tabw, pt[0], pt[1], pt[2])
    return planes.T
